# baseline (speedup 1.0000x reference)
import functools

import jax
import jax.numpy as jnp
from jax import lax
from jax.experimental import pallas as pl
from jax.experimental.pallas import tpu as pltpu
from jax.experimental.pallas import tpu_sc as plsc

GAMMA = 0.1
TAO = 10.0
B_CONST = 1.0
BETA = 1.0
LAMBDA_ = 0.1
EPS = 1e-06

K = 8192
D = 256

L = 16
NC = 2
NS = 16
NW = NC * NS
K_SC = 2048
K_TC = K - K_SC
RPW = K_SC // NW
CHUNK = 32
NCHUNK = RPW // CHUNK

BLK = 1024
NBLK = K_TC // BLK

_mesh = plsc.VectorSubcoreMesh(core_axis_name="c", subcore_axis_name="s")


@functools.partial(
    pl.kernel,
    out_type=jax.ShapeDtypeStruct((K_SC,), jnp.float32),
    mesh=_mesh,
    scratch_types=[
        pltpu.VMEM((D,), jnp.float32),
        pltpu.VMEM((CHUNK, D), jnp.float32),
        pltpu.VMEM((CHUNK, D), jnp.float32),
        pltpu.VMEM((RPW,), jnp.float32),
        pltpu.SemaphoreType.DMA,
        pltpu.SemaphoreType.DMA,
    ],
    compiler_params=pltpu.CompilerParams(needs_layout_passes=False),
)
def _sc_d2(f_hbm, p_hbm, out_hbm, c_v, buf0, buf1, out_v, sem0, sem1):
    wid = lax.axis_index("s") * NC + lax.axis_index("c")
    base = wid * RPW
    prow = K_TC + base
    pltpu.sync_copy(f_hbm, c_v)

    bufs = (buf0, buf1)
    sems = (sem0, sem1)
    copies = [None] * NCHUNK
    copies[0] = pltpu.async_copy(
        p_hbm.at[pl.ds(prow, CHUNK)], buf0, sem0)

    cvecs = [c_v[pl.ds(jb * L, L)] + EPS for jb in range(D // L)]
    lane = lax.iota(jnp.int32, L)

    for chunk in range(NCHUNK):
        buf = bufs[chunk % 2]
        if chunk + 1 < NCHUNK:
            copies[chunk + 1] = pltpu.async_copy(
                p_hbm.at[pl.ds(prow + (chunk + 1) * CHUNK, CHUNK)],
                bufs[(chunk + 1) % 2], sems[(chunk + 1) % 2])
        copies[chunk].wait()

        def group_body(g, carry, buf=buf, chunk=chunk):
            rowsums = jnp.zeros((L,), jnp.float32)
            for rr in range(L):
                row = g * L + rr
                acc = jnp.zeros((L,), jnp.float32)
                for jb in range(D // L):
                    v = buf[row, pl.ds(jb * L, L)]
                    dlt = cvecs[jb] - v
                    acc = acc + dlt * dlt
                s = jnp.sum(acc)
                rowsums = jnp.where(lane == rr, s, rowsums)
            out_v[pl.ds(chunk * CHUNK + g * L, L)] = rowsums
            return carry

        lax.fori_loop(0, CHUNK // L, group_body, 0)

    pltpu.sync_copy(out_v, out_hbm.at[pl.ds(base, RPW)])


def _tc_d2_body(c_ref, p_ref, d2_ref):
    diff = (c_ref[...] + EPS) - p_ref[...]
    sq = diff * diff
    ones = jnp.ones((D, 1), jnp.float32)
    d2_ref[...] = jax.lax.dot_general(
        sq, ones, (((1,), (0,)), ((), ())),
        preferred_element_type=jnp.float32).reshape(BLK // 128, 128)


_tc_d2 = pl.pallas_call(
    _tc_d2_body,
    grid=(NBLK,),
    in_specs=[
        pl.BlockSpec((1, D), lambda i: (0, 0)),
        pl.BlockSpec((BLK, D), lambda i: (i, 0)),
    ],
    out_specs=pl.BlockSpec((BLK // 128, 128), lambda i: (i, 0)),
    out_shape=jax.ShapeDtypeStruct((K_TC // 128, 128), jnp.float32),
)


def _softplus(z):
    return jnp.log1p(jnp.exp(z))


def _merge_body(label_ref, d2a_ref, d2b_ref, lab_ref, loss_ref, mind_ref):
    d2 = jnp.concatenate([d2a_ref[...], d2b_ref[...]], axis=0)
    mask = lab_ref[...] == label_ref[0, 0]
    d = jnp.sqrt(d2)
    e = jnp.exp(-GAMMA * d2)
    one = jnp.sum(e)
    num = jnp.sum(jnp.where(mask, e, 0.0))
    g1 = _softplus(B_CONST - (TAO - d))
    g2 = _softplus(B_CONST + (TAO - d))
    pw = jnp.sum(jnp.where(mask, g1, 0.0)) + jnp.sum(g2)
    mind2 = jnp.min(jnp.where(mask, d2, jnp.inf))
    dce = -jnp.log(num / one)
    loss_ref[0, 0] = dce + LAMBDA_ * pw
    mind_ref[0, 0] = jnp.sqrt(mind2)


_merge = pl.pallas_call(
    _merge_body,
    in_specs=[
        pl.BlockSpec(memory_space=pltpu.SMEM),
        pl.BlockSpec(memory_space=pltpu.VMEM),
        pl.BlockSpec(memory_space=pltpu.VMEM),
        pl.BlockSpec(memory_space=pltpu.VMEM),
    ],
    out_specs=(
        pl.BlockSpec(memory_space=pltpu.SMEM),
        pl.BlockSpec(memory_space=pltpu.SMEM),
    ),
    out_shape=(
        jax.ShapeDtypeStruct((1, 1), jnp.float32),
        jax.ShapeDtypeStruct((1, 1), jnp.float32),
    ),
)


def kernel(feature, label, prototypes, proto_labels):
    f = feature.astype(jnp.float32)
    d2_sc = _sc_d2(f.reshape(D), prototypes)
    d2_tc = _tc_d2(f, prototypes)
    lab = proto_labels.astype(jnp.int32).reshape(K // 128, 128)
    label2d = jnp.asarray(label, jnp.int32).reshape(1, 1)
    loss, mind = _merge(label2d, d2_tc, d2_sc.reshape(K_SC // 128, 128), lab)
    return (loss.reshape(()), mind.reshape(()))


# speedup vs baseline: 2.6004x; 2.6004x over previous
import jax
import jax.numpy as jnp
from jax.experimental import pallas as pl
from jax.experimental.pallas import tpu as pltpu

GAMMA = 0.1
TAO = 10.0
B_CONST = 1.0
BETA = 1.0
LAMBDA_ = 0.1
EPS = 1e-06

K = 8192
D = 256
CR = 512
NCH = K // CR
NBUF = 4


def _softplus(z):
    return jnp.log1p(jnp.exp(z))


def _body(label_ref, c_ref, lab_ref, p_hbm, loss_ref, mind_ref,
          buf, d2s, sems):
    copies = [None] * NCH

    def chunk_copy(i):
        return pltpu.make_async_copy(
            p_hbm.at[pl.ds(i * CR, CR)], buf.at[i % NBUF], sems.at[i % NBUF])

    for i in range(NBUF):
        copies[i] = chunk_copy(i)
        copies[i].start()

    ce = c_ref[...] + EPS
    ones = jnp.ones((D, 1), jnp.float32)
    for i in range(NCH):
        copies[i].wait()
        diff = ce - buf[i % NBUF]
        sq = diff * diff
        d2s[pl.ds(i * (CR // 128), CR // 128), :] = jax.lax.dot_general(
            sq, ones, (((1,), (0,)), ((), ())),
            preferred_element_type=jnp.float32).reshape(CR // 128, 128)
        if i + NBUF < NCH:
            copies[i + NBUF] = chunk_copy(i + NBUF)
            copies[i + NBUF].start()

    d2 = d2s[...]
    mask = lab_ref[...] == label_ref[0, 0]
    d = jnp.sqrt(d2)
    e = jnp.exp(-GAMMA * d2)
    one = jnp.sum(e)
    num = jnp.sum(jnp.where(mask, e, 0.0))
    g1 = _softplus(B_CONST - (TAO - d))
    g2 = _softplus(B_CONST + (TAO - d))
    pw = jnp.sum(jnp.where(mask, g1, 0.0)) + jnp.sum(g2)
    mind2 = jnp.min(jnp.where(mask, d2, jnp.inf))
    dce = -jnp.log(num / one)
    loss_ref[0, 0] = dce + LAMBDA_ * pw
    mind_ref[0, 0] = jnp.sqrt(mind2)


_tc_full = pl.pallas_call(
    _body,
    in_specs=[
        pl.BlockSpec(memory_space=pltpu.SMEM),
        pl.BlockSpec(memory_space=pltpu.VMEM),
        pl.BlockSpec(memory_space=pltpu.VMEM),
        pl.BlockSpec(memory_space=pl.ANY),
    ],
    out_specs=(
        pl.BlockSpec(memory_space=pltpu.SMEM),
        pl.BlockSpec(memory_space=pltpu.SMEM),
    ),
    out_shape=(
        jax.ShapeDtypeStruct((1, 1), jnp.float32),
        jax.ShapeDtypeStruct((1, 1), jnp.float32),
    ),
    scratch_shapes=[
        pltpu.VMEM((NBUF, CR, D), jnp.float32),
        pltpu.VMEM((K // 128, 128), jnp.float32),
        pltpu.SemaphoreType.DMA((NBUF,)),
    ],
)


def kernel(feature, label, prototypes, proto_labels):
    lab = proto_labels.astype(jnp.int32).reshape(K // 128, 128)
    label2d = jnp.asarray(label, jnp.int32).reshape(1, 1)
    loss, mind = _tc_full(label2d, feature.astype(jnp.float32), lab, prototypes)
    return (loss.reshape(()), mind.reshape(()))
